# baseline (speedup 1.0000x reference)
import functools

import jax
import jax.numpy as jnp
from jax import lax
from jax.experimental import pallas as pl
from jax.experimental.pallas import tpu as pltpu
from jax.experimental.pallas import tpu_sc as plsc

_D = 77
_N = 4096
_V = 1000
_TT = 1024
_NC, _NS = 2, 16
_NW = _NC * _NS
_IW = _N // _NW
_NG = _IW // 16

_mesh = plsc.VectorSubcoreMesh(core_axis_name="c", subcore_axis_name="s")


@functools.partial(
    pl.kernel,
    out_type=jax.ShapeDtypeStruct((_D, _D, _N), jnp.float32),
    mesh=_mesh,
    scratch_types=[
        pltpu.VMEM((_D * _TT,), jnp.float32),
        pltpu.VMEM((_D, _IW), jnp.int32),
        pltpu.VMEM((2, _D, _IW), jnp.float32),
        pltpu.SemaphoreType.DMA,
        pltpu.SemaphoreType.DMA,
    ],
    compiler_params=pltpu.CompilerParams(needs_layout_passes=False),
)
def _embed(xt_hbm, tablet_hbm, out_hbm, tablet_v, idx_v, bufs, sem0, sem1):
    wsem = (sem0, sem1)
    wid = lax.axis_index("s") * _NC + lax.axis_index("c")
    io = wid * _IW
    pltpu.sync_copy(tablet_hbm, tablet_v)
    pltpu.sync_copy(xt_hbm.at[:, pl.ds(io, _IW)], idx_v)

    def build(j, b):
        tv = [idx_v[j, pl.ds(16 * g, 16)] for g in range(_NG)]
        lanes = [lax.iota(jnp.int32, 16) + 16 * g for g in range(_NG)]
        buf = bufs.at[b]
        zero = jnp.zeros((16,), jnp.int32)

        @plsc.parallel_loop(0, _D, step=11)
        def _cgrp(c0):
            for cc in range(11):
                c = c0 + cc
                cs = zero + c
                coff = c * _TT
                for g in range(_NG):
                    vals = plsc.load_gather(tablet_v, [tv[g] + coff])
                    plsc.store_scatter(buf, [cs, lanes[g]], vals)

    def fire(j, b):
        pltpu.async_copy(
            bufs.at[b], out_hbm.at[j, :, pl.ds(io, _IW)], wsem[b]
        )

    def drain(b):
        pltpu.make_async_copy(
            bufs.at[b], out_hbm.at[0, :, pl.ds(io, _IW)], wsem[b]
        ).wait()

    for b in range(2):
        build(b, b)
        fire(b, b)

    @pl.loop(0, _D - 3, step=2)
    def _group(k0):
        for b in range(2):
            kn = k0 + b + 2
            drain(b)
            build(kn, b)
            fire(kn, b)

    drain(0)
    build(_D - 1, 0)
    fire(_D - 1, 0)
    drain(0)
    drain(1)


def kernel(x, table):
    xt = x.T.astype(jnp.int32)
    tablet = jnp.pad(table.T, ((0, 0), (0, _TT - _V))).reshape(-1)
    out_t = _embed(xt, tablet)
    return jnp.transpose(out_t, (2, 0, 1))


# speedup vs baseline: 1.5115x; 1.5115x over previous
import functools

import jax
import jax.numpy as jnp
from jax import lax
from jax.experimental import pallas as pl
from jax.experimental.pallas import tpu as pltpu
from jax.experimental.pallas import tpu_sc as plsc

_D = 77
_N = 4096
_V = 1000
_TT = 1024
_NC, _NS = 2, 16
_NW = _NC * _NS
_IW = _N // _NW
_NG = _IW // 16

_mesh = plsc.VectorSubcoreMesh(core_axis_name="c", subcore_axis_name="s")


@functools.partial(
    pl.kernel,
    out_type=jax.ShapeDtypeStruct((_D, _D, _N), jnp.float32),
    mesh=_mesh,
    scratch_types=[
        pltpu.VMEM((_D * _TT,), jnp.float32),
        pltpu.VMEM((_D, _IW), jnp.int32),
        pltpu.VMEM((2, _D, _IW), jnp.float32),
        pltpu.SemaphoreType.DMA,
        pltpu.SemaphoreType.DMA,
    ],
    compiler_params=pltpu.CompilerParams(needs_layout_passes=False),
)
def _embed(xt_hbm, tablet_hbm, out_hbm, tablet_v, idx_v, bufs, sem0, sem1):
    wsem = (sem0, sem1)
    wid = lax.axis_index("s") * _NC + lax.axis_index("c")
    io = wid * _IW
    pltpu.sync_copy(tablet_hbm, tablet_v)
    pltpu.sync_copy(xt_hbm.at[:, pl.ds(io, _IW)], idx_v)

    def build(j, b):
        tv = [idx_v[j, pl.ds(16 * g, 16)] for g in range(_NG)]
        lanes = [lax.iota(jnp.int32, 16) + 16 * g for g in range(_NG)]
        buf = bufs.at[b]
        zero = jnp.zeros((16,), jnp.int32)

        @plsc.parallel_loop(0, _D, step=7)
        def _cgrp(c0):
            for cc in range(7):
                c = c0 + cc
                cs = zero + c
                coff = c * _TT
                for g in range(_NG):
                    vals = plsc.load_gather(tablet_v, [tv[g] + coff])
                    plsc.store_scatter(buf, [cs, lanes[g]], vals)

    def fire(j, b):
        pltpu.async_copy(
            bufs.at[b], out_hbm.at[j, :, pl.ds(io, _IW)], wsem[b]
        )

    def drain(b):
        pltpu.make_async_copy(
            bufs.at[b], out_hbm.at[0, :, pl.ds(io, _IW)], wsem[b]
        ).wait()

    for b in range(2):
        build(b, b)
        fire(b, b)

    @pl.loop(0, _D - 3, step=2)
    def _group(k0):
        for b in range(2):
            kn = k0 + b + 2
            drain(b)
            build(kn, b)
            fire(kn, b)

    drain(0)
    build(_D - 1, 0)
    fire(_D - 1, 0)
    drain(0)
    drain(1)


def kernel(x, table):
    xt = x.T.astype(jnp.int32)
    tablet = jnp.pad(table.T, ((0, 0), (0, _TT - _V))).reshape(-1)
    out_t = _embed(xt, tablet)
    return jnp.transpose(out_t, (2, 0, 1))
